# baseline (speedup 1.0000x reference)
import functools

import jax
import jax.numpy as jnp
from jax import lax
from jax.experimental import pallas as pl
from jax.experimental.pallas import tpu as pltpu
from jax.experimental.pallas import tpu_sc as plsc

VOCAB = 10000
D = 64
BATCH = 4096
SEQ = 200

NC = 2
NS = 16
NW = NC * NS

BB = BATCH // NW
SP = 2
NBLK = SEQ // SP
NB = 2
PAD = 8

_mesh = plsc.VectorSubcoreMesh(
    core_axis_name="c", subcore_axis_name="s", num_cores=NC, num_subcores=NS
)


@functools.partial(
    pl.kernel,
    out_type=jax.ShapeDtypeStruct((SEQ, D // 8, BATCH // BB, 8, BB), jnp.float32),
    mesh=_mesh,
    scratch_types=[
        pltpu.VMEM((BB, SEQ), jnp.int32),
        pltpu.VMEM((SEQ, BB), jnp.int32),
        [pltpu.VMEM((SP * BB, D), jnp.float32)] * NB,
        [pltpu.VMEM((SP, D // 8, 8, BB + PAD), jnp.float32)] * NB,
        pltpu.SemaphoreType.DMA,
        pltpu.SemaphoreType.DMA,
    ],
    compiler_params=pltpu.CompilerParams(
        use_tc_tiling_on_sc=False, needs_layout_passes=False
    ),
)
def _embed_lookup(idx_hbm, table_hbm, out_hbm, idx_v, idx_vt, gbufs, tbufs,
                  gsem, osem):
    wid = lax.axis_index("s") * NC + lax.axis_index("c")
    wbase = wid * BB

    pltpu.sync_copy(idx_hbm.at[pl.ds(wbase, BB)], idx_v)
    iota = lax.iota(jnp.int32, 16)
    rows = [iota + 16 * k for k in range(BB // 16)]
    dtv = [(iota + 16 * k) // 8 for k in range(D // 16)]
    drv = [(iota + 16 * k) % 8 for k in range(D // 16)]

    def transpose_idx(lo, hi):
        @plsc.parallel_loop(lo, hi, unroll=4)
        def _(s):
            col = jnp.full((16,), s, jnp.int32)
            for seg in range(BB // 16):
                v = plsc.load_gather(idx_v, [rows[seg], col])
                idx_vt[s, pl.ds(seg * 16, 16)] = v

    def gather(blk, slot):
        for e in range(SP):
            pltpu.async_copy(
                table_hbm.at[idx_vt.at[blk * SP + e]],
                gbufs[slot].at[pl.ds(e * BB, BB)],
                gsem,
            )

    transpose_idx(0, SP * NB)
    for b in range(NB):
        gather(b, b)
    transpose_idx(SP * NB, SEQ)

    @pl.loop(0, NBLK, step=NB)
    def _(g):
        for b in range(NB):
            blk = g + b
            pltpu.make_async_copy(
                table_hbm.at[idx_vt.at[0]], gbufs[b], gsem
            ).wait()

            @pl.when(blk >= NB)
            def _():
                pltpu.make_async_copy(
                    tbufs[b].at[:, :, :, pl.ds(0, BB)],
                    out_hbm.at[pl.ds(0, SP), slice(None), 0],
                    osem,
                ).wait()

            for e in range(SP):
                @plsc.parallel_loop(0, BB, unroll=16)
                def _(bc):
                    bcv = jnp.full((16,), bc, jnp.int32)
                    vs = [
                        gbufs[b][e * BB + bc, pl.ds(16 * k, 16)]
                        for k in range(D // 16)
                    ]
                    for k in range(D // 16):
                        plsc.store_scatter(
                            tbufs[b].at[e], [dtv[k], drv[k], bcv], vs[k]
                        )

            pltpu.async_copy(
                tbufs[b].at[:, :, :, pl.ds(0, BB)],
                out_hbm.at[pl.ds(blk * SP, SP), slice(None), wid],
                osem,
            )

            @pl.when(blk + NB < NBLK)
            def _():
                gather(blk + NB, b)

    for b in range(NB):
        pltpu.make_async_copy(
            tbufs[b].at[:, :, :, pl.ds(0, BB)],
            out_hbm.at[pl.ds(0, SP), slice(None), 0],
            osem,
        ).wait()


def kernel(token_ids, embedding_table):
    x = _embed_lookup(jnp.asarray(token_ids, jnp.int32), embedding_table)
    return x.transpose(2, 4, 0, 1, 3).reshape(BATCH, SEQ, D)


# speedup vs baseline: 1.1242x; 1.1242x over previous
import functools

import jax
import jax.numpy as jnp
from jax import lax
from jax.experimental import pallas as pl
from jax.experimental.pallas import tpu as pltpu
from jax.experimental.pallas import tpu_sc as plsc

VOCAB = 10000
D = 64
BATCH = 4096
SEQ = 200

NC = 2
NS = 16
NW = NC * NS

BB = BATCH // NW
SP = 2
NBLK = SEQ // SP
NB = 2
PAD = 8

_mesh = plsc.VectorSubcoreMesh(
    core_axis_name="c", subcore_axis_name="s", num_cores=NC, num_subcores=NS
)


@functools.partial(
    pl.kernel,
    out_type=jax.ShapeDtypeStruct((SEQ, D // 8, BATCH // BB, 8, BB), jnp.float32),
    mesh=_mesh,
    scratch_types=[
        pltpu.VMEM((BB, SEQ), jnp.int32),
        pltpu.VMEM((SEQ, BB), jnp.int32),
        [pltpu.VMEM((SP * BB, D), jnp.float32)] * NB,
        [pltpu.VMEM((SP, D // 8, 8, BB + PAD), jnp.float32)] * NB,
        pltpu.SemaphoreType.DMA,
        pltpu.SemaphoreType.DMA,
    ],
    compiler_params=pltpu.CompilerParams(
        use_tc_tiling_on_sc=False, needs_layout_passes=False
    ),
)
def _embed_lookup(idx_hbm, table_hbm, out_hbm, idx_v, idx_vt, gbufs, tbufs,
                  gsem, osem):
    wid = lax.axis_index("s") * NC + lax.axis_index("c")
    wbase = wid * BB

    pltpu.sync_copy(idx_hbm.at[pl.ds(wbase, BB)], idx_v)
    iota = lax.iota(jnp.int32, 16)
    rows = [iota + 16 * k for k in range(BB // 16)]
    dtv = [(iota + 16 * k) // 8 for k in range(D // 16)]
    drv = [(iota + 16 * k) % 8 for k in range(D // 16)]

    def transpose_idx(lo, hi):
        @plsc.parallel_loop(lo, hi, unroll=4)
        def _(s):
            col = jnp.full((16,), s, jnp.int32)
            for seg in range(BB // 16):
                v = plsc.load_gather(idx_v, [rows[seg], col])
                idx_vt[s, pl.ds(seg * 16, 16)] = v

    def gather(blk, slot):
        for e in range(SP):
            pltpu.async_copy(
                table_hbm.at[idx_vt.at[blk * SP + e]],
                gbufs[slot].at[pl.ds(e * BB, BB)],
                gsem,
            )

    transpose_idx(0, SP * NB)
    for b in range(NB):
        gather(b, b)
    transpose_idx(SP * NB, SEQ)

    @pl.loop(0, NBLK, step=NB)
    def _(g):
        for b in range(NB):
            blk = g + b
            pltpu.make_async_copy(
                table_hbm.at[idx_vt.at[0]], gbufs[b], gsem
            ).wait()

            @pl.when(blk >= NB)
            def _():
                pltpu.make_async_copy(
                    tbufs[b].at[:, :, :, pl.ds(0, BB)],
                    out_hbm.at[pl.ds(0, SP), slice(None), 0],
                    osem,
                ).wait()

            for e in range(SP):
                @plsc.parallel_loop(0, BB, unroll=8)
                def _(bc):
                    bcv = jnp.full((16,), bc, jnp.int32)
                    vs = [
                        gbufs[b][e * BB + bc, pl.ds(16 * k, 16)]
                        for k in range(D // 16)
                    ]
                    for k in range(D // 16):
                        plsc.store_scatter(
                            tbufs[b].at[e], [dtv[k], drv[k], bcv], vs[k]
                        )

            pltpu.async_copy(
                tbufs[b].at[:, :, :, pl.ds(0, BB)],
                out_hbm.at[pl.ds(blk * SP, SP), slice(None), wid],
                osem,
            )

            @pl.when(blk + NB < NBLK)
            def _():
                gather(blk + NB, b)

    for b in range(NB):
        pltpu.make_async_copy(
            tbufs[b].at[:, :, :, pl.ds(0, BB)],
            out_hbm.at[pl.ds(0, SP), slice(None), 0],
            osem,
        ).wait()


def kernel(token_ids, embedding_table):
    x = _embed_lookup(jnp.asarray(token_ids, jnp.int32), embedding_table)
    return x.transpose(2, 4, 0, 1, 3).reshape(BATCH, SEQ, D)


# speedup vs baseline: 1.1356x; 1.0101x over previous
import functools

import jax
import jax.numpy as jnp
from jax import lax
from jax.experimental import pallas as pl
from jax.experimental.pallas import tpu as pltpu
from jax.experimental.pallas import tpu_sc as plsc

VOCAB = 10000
D = 64
BATCH = 4096
SEQ = 200

NC = 2
NS = 16
NW = NC * NS

BB = BATCH // NW
SP = 2
NBLK = SEQ // SP
NB = 2
PAD = 8

_mesh = plsc.VectorSubcoreMesh(
    core_axis_name="c", subcore_axis_name="s", num_cores=NC, num_subcores=NS
)


@functools.partial(
    pl.kernel,
    out_type=jax.ShapeDtypeStruct((SEQ, D // 8, BATCH // BB, 8, BB), jnp.float32),
    mesh=_mesh,
    scratch_types=[
        pltpu.VMEM((BB, SEQ), jnp.int32),
        pltpu.VMEM((SEQ, BB), jnp.int32),
        [pltpu.VMEM((SP * BB, D), jnp.float32)] * NB,
        [pltpu.VMEM((SP, D // 8, 8, BB + PAD), jnp.float32)] * NB,
        pltpu.SemaphoreType.DMA,
        pltpu.SemaphoreType.DMA,
    ],
    compiler_params=pltpu.CompilerParams(
        use_tc_tiling_on_sc=False, needs_layout_passes=False
    ),
)
def _embed_lookup(idx_hbm, table_hbm, out_hbm, idx_v, idx_vt, gbufs, tbufs,
                  gsem, osem):
    wid = lax.axis_index("s") * NC + lax.axis_index("c")
    wbase = wid * BB

    pltpu.sync_copy(idx_hbm.at[pl.ds(wbase, BB)], idx_v)
    iota = lax.iota(jnp.int32, 16)
    rows = [iota + 16 * k for k in range(BB // 16)]
    dtv = [(iota + 16 * k) // 8 for k in range(D // 16)]
    drv = [(iota + 16 * k) % 8 for k in range(D // 16)]

    def transpose_idx(lo, hi):
        @plsc.parallel_loop(lo, hi, unroll=4)
        def _(s):
            col = jnp.full((16,), s, jnp.int32)
            for seg in range(BB // 16):
                v = plsc.load_gather(idx_v, [rows[seg], col])
                idx_vt[s, pl.ds(seg * 16, 16)] = v

    def gather(blk, slot):
        for e in range(SP):
            pltpu.async_copy(
                table_hbm.at[idx_vt.at[blk * SP + e]],
                gbufs[slot].at[pl.ds(e * BB, BB)],
                gsem,
            )

    transpose_idx(0, SP * NB)
    for b in range(NB):
        gather(b, b)
    transpose_idx(SP * NB, SEQ)

    @pl.loop(0, NBLK, step=NB)
    def _(g):
        for b in range(NB):
            blk = g + b

            @pl.when(blk >= NB)
            def _():
                pltpu.make_async_copy(
                    tbufs[b].at[:, :, :, pl.ds(0, BB)],
                    out_hbm.at[pl.ds(0, SP), slice(None), 0],
                    osem,
                ).wait()

            for e in range(SP):
                pltpu.make_async_copy(
                    table_hbm.at[idx_vt.at[0]],
                    gbufs[b].at[pl.ds(0, BB)],
                    gsem,
                ).wait()

                @plsc.parallel_loop(0, BB, unroll=8)
                def _(bc):
                    bcv = jnp.full((16,), bc, jnp.int32)
                    vs = [
                        gbufs[b][e * BB + bc, pl.ds(16 * k, 16)]
                        for k in range(D // 16)
                    ]
                    for k in range(D // 16):
                        plsc.store_scatter(
                            tbufs[b].at[e], [dtv[k], drv[k], bcv], vs[k]
                        )

            pltpu.async_copy(
                tbufs[b].at[:, :, :, pl.ds(0, BB)],
                out_hbm.at[pl.ds(blk * SP, SP), slice(None), wid],
                osem,
            )

            @pl.when(blk + NB < NBLK)
            def _():
                gather(blk + NB, b)

    for b in range(NB):
        pltpu.make_async_copy(
            tbufs[b].at[:, :, :, pl.ds(0, BB)],
            out_hbm.at[pl.ds(0, SP), slice(None), 0],
            osem,
        ).wait()


def kernel(token_ids, embedding_table):
    x = _embed_lookup(jnp.asarray(token_ids, jnp.int32), embedding_table)
    return x.transpose(2, 4, 0, 1, 3).reshape(BATCH, SEQ, D)
